# baseline (speedup 1.0000x reference)
import jax
import jax.numpy as jnp
from jax.experimental import pallas as pl
from jax.experimental.pallas import tpu as pltpu

BATCH = 4096
SEQ = 500
EMB = 64
VOCAB = 1000

B_TILE = 512
K_TILE = 3200


def _mlp_body(x_ref, w1_ref, b1_ref, w2_ref, b2_ref, w3_ref, b3_ref,
              mu_ref, sc_ref, acc_ref):
    k = pl.program_id(1)
    nk = pl.num_programs(1)

    @pl.when(k == 0)
    def _():
        acc_ref[...] = jnp.zeros_like(acc_ref)

    acc_ref[...] += jnp.dot(x_ref[...], w1_ref[...],
                            preferred_element_type=jnp.float32)

    @pl.when(k == nk - 1)
    def _():
        h1 = jnp.maximum(acc_ref[...] + b1_ref[...], 0.0)
        h2 = jnp.maximum(
            jnp.dot(h1, w2_ref[...], preferred_element_type=jnp.float32)
            + b2_ref[...], 0.0)
        out = jnp.dot(h2, w3_ref[...],
                      preferred_element_type=jnp.float32) + b3_ref[...]
        mu_ref[...] = out[:, :2]
        rho = out[:, 2:] - 5.0
        sc_ref[...] = jnp.logaddexp(rho, 0.0)


def _mlp(x, W1, b1, W2, b2, W3, b3):
    nb = BATCH // B_TILE
    nk = (SEQ * EMB) // K_TILE
    grid = (nb, nk)
    mu, scale = pl.pallas_call(
        _mlp_body,
        grid=grid,
        in_specs=[
            pl.BlockSpec((B_TILE, K_TILE), lambda b, k: (b, k)),
            pl.BlockSpec((K_TILE, 128), lambda b, k: (k, 0)),
            pl.BlockSpec((128,), lambda b, k: (0,)),
            pl.BlockSpec((128, 64), lambda b, k: (0, 0)),
            pl.BlockSpec((64,), lambda b, k: (0,)),
            pl.BlockSpec((64, 4), lambda b, k: (0, 0)),
            pl.BlockSpec((4,), lambda b, k: (0,)),
        ],
        out_specs=[
            pl.BlockSpec((B_TILE, 2), lambda b, k: (b, 0)),
            pl.BlockSpec((B_TILE, 2), lambda b, k: (b, 0)),
        ],
        out_shape=[
            jax.ShapeDtypeStruct((BATCH, 2), jnp.float32),
            jax.ShapeDtypeStruct((BATCH, 2), jnp.float32),
        ],
        scratch_shapes=[pltpu.VMEM((B_TILE, 128), jnp.float32)],
    )(x, W1, b1, W2, b2, W3, b3)
    return mu, scale


def kernel(data, E, W1, b1, W2, b2, W3, b3):
    x = jnp.take(E, data, axis=0).reshape(BATCH, SEQ * EMB)
    return _mlp(x, W1, b1, W2, b2, W3, b3)


# speedup vs baseline: 7.6977x; 7.6977x over previous
import functools

import jax
import jax.numpy as jnp
from jax import lax
from jax.experimental import pallas as pl
from jax.experimental.pallas import tpu as pltpu
from jax.experimental.pallas import tpu_sc as plsc

BATCH = 4096
SEQ = 500
EMB = 64
VOCAB = 1000
H1 = 128

SEQ_PAD = 512
N_CHUNK = SEQ_PAD // 128
NW = 32
B_PER_W = BATCH // NW


def _ptab_body(e_ref, w1_ref, p_ref):
    for i in range(4):
        p_ref[i] = jnp.dot(e_ref[...], w1_ref[i],
                           preferred_element_type=jnp.float32)


def _ptab(E, W1r):
    return pl.pallas_call(
        _ptab_body,
        grid=(SEQ // 4,),
        in_specs=[
            pl.BlockSpec((VOCAB, EMB), lambda k: (0, 0)),
            pl.BlockSpec((4, EMB, H1), lambda k: (k, 0, 0)),
        ],
        out_specs=pl.BlockSpec((4, VOCAB, H1), lambda k: (k, 0, 0)),
        out_shape=jax.ShapeDtypeStruct((SEQ, VOCAB, H1), jnp.float32),
    )(E, W1r)


def _sc_body(data_hbm, p_hbm, out_hbm, dbuf, i0, i1, i2, i3, gbuf, abuf, sem):
    ibufs = (i0, i1, i2, i3)
    wid = lax.axis_index("s") * 2 + lax.axis_index("c")
    base = wid * B_PER_W

    def row_body(b, _):
        pltpu.sync_copy(data_hbm.at[base + b], dbuf)
        lane = lax.iota(jnp.int32, 16)
        for c in range(SEQ_PAD // 16):
            v = dbuf[pl.ds(c * 16, 16)] + (c * 16 + lane) * VOCAB
            if (c + 1) * 16 > SEQ:
                v = jnp.where(lane < SEQ - c * 16, v, 0)
            ibufs[c // 8][pl.ds((c % 8) * 16, 16)] = v
        copies = []
        for r in range(N_CHUNK):
            copies.append(pltpu.async_copy(
                p_hbm.at[ibufs[r]], gbuf.at[pl.ds(r * 128, 128)], sem))
        for cp in copies:
            cp.wait()

        def acc_body(i, carry):
            out = []
            for j in range(8):
                a = carry[j]
                for u in range(4):
                    a = a + gbuf[i * 4 + u, pl.ds(j * 16, 16)]
                out.append(a)
            return tuple(out)

        zeros = tuple(jnp.zeros((16,), jnp.float32) for _ in range(8))
        accs = lax.fori_loop(0, SEQ // 4, acc_body, zeros)
        for j in range(8):
            abuf[pl.ds(j * 16, 16)] = accs[j]
        pltpu.sync_copy(abuf, out_hbm.at[base + b])
        return ()

    lax.fori_loop(0, B_PER_W, row_body, ())


def _sc_gather(data_p, P):
    mesh = plsc.VectorSubcoreMesh(core_axis_name="c", subcore_axis_name="s")
    f = functools.partial(
        pl.kernel,
        mesh=mesh,
        out_type=jax.ShapeDtypeStruct((BATCH, H1), jnp.float32),
        scratch_types=[
            pltpu.VMEM((SEQ_PAD,), jnp.int32),
            pltpu.VMEM((128,), jnp.int32),
            pltpu.VMEM((128,), jnp.int32),
            pltpu.VMEM((128,), jnp.int32),
            pltpu.VMEM((128,), jnp.int32),
            pltpu.VMEM((SEQ_PAD, H1), jnp.float32),
            pltpu.VMEM((H1,), jnp.float32),
            pltpu.SemaphoreType.DMA,
        ],
    )(_sc_body)
    return f(data_p, P)


def _epi_body(x_ref, b1_ref, w2_ref, b2_ref, w3_ref, b3_ref, mu_ref, sc_ref):
    h1 = jnp.maximum(x_ref[...] + b1_ref[...], 0.0)
    h2 = jnp.maximum(
        jnp.dot(h1, w2_ref[...], preferred_element_type=jnp.float32)
        + b2_ref[...], 0.0)
    out = jnp.dot(h2, w3_ref[...],
                  preferred_element_type=jnp.float32) + b3_ref[...]
    mu_ref[...] = out[:, :2]
    sc_ref[...] = jnp.logaddexp(out[:, 2:] - 5.0, 0.0)


def _epi(x, b1, W2, b2, W3, b3):
    B_TILE = 512
    return pl.pallas_call(
        _epi_body,
        grid=(BATCH // B_TILE,),
        in_specs=[
            pl.BlockSpec((B_TILE, H1), lambda b: (b, 0)),
            pl.BlockSpec((H1,), lambda b: (0,)),
            pl.BlockSpec((H1, 64), lambda b: (0, 0)),
            pl.BlockSpec((64,), lambda b: (0,)),
            pl.BlockSpec((64, 4), lambda b: (0, 0)),
            pl.BlockSpec((4,), lambda b: (0,)),
        ],
        out_specs=[
            pl.BlockSpec((B_TILE, 2), lambda b: (b, 0)),
            pl.BlockSpec((B_TILE, 2), lambda b: (b, 0)),
        ],
        out_shape=[
            jax.ShapeDtypeStruct((BATCH, 2), jnp.float32),
            jax.ShapeDtypeStruct((BATCH, 2), jnp.float32),
        ],
    )(x, b1, W2, b2, W3, b3)


def kernel(data, E, W1, b1, W2, b2, W3, b3):
    W1r = W1.reshape(SEQ, EMB, H1)
    P = _ptab(E, W1r).reshape(SEQ * VOCAB, H1)
    data_p = jnp.pad(data, ((0, 0), (0, SEQ_PAD - SEQ)))
    out1 = _sc_gather(data_p, P)
    return _epi(out1, b1, W2, b2, W3, b3)


# speedup vs baseline: 7.7393x; 1.0054x over previous
import functools

import jax
import jax.numpy as jnp
from jax import lax
from jax.experimental import pallas as pl
from jax.experimental.pallas import tpu as pltpu
from jax.experimental.pallas import tpu_sc as plsc

BATCH = 4096
SEQ = 500
EMB = 64
VOCAB = 1000
H1 = 128

SEQ_PAD = 512
N_CHUNK = SEQ_PAD // 128
ZERO_ROW = SEQ * VOCAB
NW = 32
B_PER_W = BATCH // NW
GROUP = 32


def _ptab_body(e_ref, w1_ref, p_ref):
    k = pl.program_id(0)

    @pl.when(k < SEQ // 4)
    def _():
        for i in range(4):
            p_ref[i] = jnp.dot(e_ref[...], w1_ref[i],
                               preferred_element_type=jnp.float32)

    @pl.when(k == SEQ // 4)
    def _():
        p_ref[...] = jnp.zeros_like(p_ref)


def _ptab(E, W1r):
    return pl.pallas_call(
        _ptab_body,
        grid=(SEQ // 4 + 1,),
        in_specs=[
            pl.BlockSpec((VOCAB, EMB), lambda k: (0, 0)),
            pl.BlockSpec((4, EMB, H1),
                         lambda k: (jnp.minimum(k, SEQ // 4 - 1), 0, 0)),
        ],
        out_specs=pl.BlockSpec((4, VOCAB, H1), lambda k: (k, 0, 0)),
        out_shape=jax.ShapeDtypeStruct((SEQ + 4, VOCAB, H1), jnp.float32),
    )(E, W1r)


def _compute_idx(dbuf, ibuf, row_local):
    lane = lax.iota(jnp.int32, 16)
    for c in range(SEQ_PAD // 16):
        v = dbuf[row_local, pl.ds(c * 16, 16)] + (c * 16 + lane) * VOCAB
        if (c + 1) * 16 > SEQ:
            v = jnp.where(lane < SEQ - c * 16, v, ZERO_ROW)
        ibuf[c // 8, pl.ds((c % 8) * 16, 16)] = v


def _fire(p_hbm, ibuf, c, gbufs, sems):
    return pltpu.async_copy(p_hbm.at[ibuf.at[c]], gbufs[c % 2], sems[c % 2])


def _acc_chunk(gbuf, accs):
    def body(i, carry):
        out = []
        for j in range(8):
            a = carry[j]
            for u in range(4):
                a = a + gbuf[i * 4 + u, pl.ds(j * 16, 16)]
            out.append(a)
        return tuple(out)

    return lax.fori_loop(0, 32, body, accs)


def _store_row(obuf, row_local, accs):
    for j in range(8):
        obuf[row_local, pl.ds(j * 16, 16)] = accs[j]


def _sc_body(data_hbm, p_hbm, out_hbm, dbuf, ibufA, ibufB, gb0, gb1, obuf,
             sem0, sem1):
    gbufs = (gb0, gb1)
    sems = (sem0, sem1)
    wid = lax.axis_index("s") * 2 + lax.axis_index("c")
    base = wid * B_PER_W

    def wait_chunk(ibuf, c):
        pltpu.make_async_copy(
            p_hbm.at[ibuf.at[c]], gbufs[c % 2], sems[c % 2]).wait()

    def do_row(b_local, ibuf_cur, ibuf_next, late_next):
        accs = tuple(jnp.zeros((16,), jnp.float32) for _ in range(8))
        for c in range(N_CHUNK):
            if c < N_CHUNK - 1:
                _fire(p_hbm, ibuf_cur, c + 1, gbufs, sems)
            elif not late_next:
                _compute_idx(dbuf, ibuf_next, (b_local + 1) % GROUP)
                _fire(p_hbm, ibuf_next, 0, gbufs, sems)
            wait_chunk(ibuf_cur, c)
            accs = _acc_chunk(gbufs[c % 2], accs)
        _store_row(obuf, b_local % GROUP, accs)

    def pair_body(k, carry):
        do_row(2 * k, ibufA, ibufB, late_next=False)
        do_row(2 * k + 1, ibufB, ibufA, late_next=False)
        return carry

    pltpu.sync_copy(data_hbm.at[pl.ds(base, GROUP)], dbuf)
    _compute_idx(dbuf, ibufA, 0)
    _fire(p_hbm, ibufA, 0, gbufs, sems)
    half = GROUP // 2
    for g in range(B_PER_W // GROUP):
        lax.fori_loop(g * half, (g + 1) * half - 1, pair_body, 0)
        b1 = g * GROUP + GROUP - 1
        do_row(b1 - 1, ibufA, ibufB, late_next=False)
        do_row(b1, ibufB, ibufA, late_next=True)
        pltpu.sync_copy(obuf,
                        out_hbm.at[pl.ds(base + g * GROUP, GROUP)])
        if g + 1 < B_PER_W // GROUP:
            pltpu.sync_copy(
                data_hbm.at[pl.ds(base + (g + 1) * GROUP, GROUP)], dbuf)
            _compute_idx(dbuf, ibufA, 0)
            _fire(p_hbm, ibufA, 0, gbufs, sems)


def _sc_gather(data_p, P):
    mesh = plsc.VectorSubcoreMesh(core_axis_name="c", subcore_axis_name="s")
    f = functools.partial(
        pl.kernel,
        mesh=mesh,
        out_type=jax.ShapeDtypeStruct((BATCH, H1), jnp.float32),
        scratch_types=[
            pltpu.VMEM((GROUP, SEQ_PAD), jnp.int32),
            pltpu.VMEM((N_CHUNK, 128), jnp.int32),
            pltpu.VMEM((N_CHUNK, 128), jnp.int32),
            pltpu.VMEM((128, H1), jnp.float32),
            pltpu.VMEM((128, H1), jnp.float32),
            pltpu.VMEM((GROUP, H1), jnp.float32),
            pltpu.SemaphoreType.DMA,
            pltpu.SemaphoreType.DMA,
        ],
    )(_sc_body)
    return f(data_p, P)


def _epi_body(x_ref, b1_ref, w2_ref, b2_ref, w3_ref, b3_ref, mu_ref, sc_ref):
    h1 = jnp.maximum(x_ref[...] + b1_ref[...], 0.0)
    h2 = jnp.maximum(
        jnp.dot(h1, w2_ref[...], preferred_element_type=jnp.float32)
        + b2_ref[...], 0.0)
    out = jnp.dot(h2, w3_ref[...],
                  preferred_element_type=jnp.float32) + b3_ref[...]
    mu_ref[...] = out[:, :2]
    sc_ref[...] = jnp.logaddexp(out[:, 2:] - 5.0, 0.0)


def _epi(x, b1, W2, b2, W3, b3):
    B_TILE = 512
    return pl.pallas_call(
        _epi_body,
        grid=(BATCH // B_TILE,),
        in_specs=[
            pl.BlockSpec((B_TILE, H1), lambda b: (b, 0)),
            pl.BlockSpec((H1,), lambda b: (0,)),
            pl.BlockSpec((H1, 64), lambda b: (0, 0)),
            pl.BlockSpec((64,), lambda b: (0,)),
            pl.BlockSpec((64, 4), lambda b: (0, 0)),
            pl.BlockSpec((4,), lambda b: (0,)),
        ],
        out_specs=[
            pl.BlockSpec((B_TILE, 2), lambda b: (b, 0)),
            pl.BlockSpec((B_TILE, 2), lambda b: (b, 0)),
        ],
        out_shape=[
            jax.ShapeDtypeStruct((BATCH, 2), jnp.float32),
            jax.ShapeDtypeStruct((BATCH, 2), jnp.float32),
        ],
    )(x, b1, W2, b2, W3, b3)


def kernel(data, E, W1, b1, W2, b2, W3, b3):
    W1r = W1.reshape(SEQ, EMB, H1)
    P = _ptab(E, W1r).reshape((SEQ + 4) * VOCAB, H1)
    data_p = jnp.pad(data, ((0, 0), (0, SEQ_PAD - SEQ)))
    out1 = _sc_gather(data_p, P)
    return _epi(out1, b1, W2, b2, W3, b3)


# speedup vs baseline: 11.8286x; 1.5284x over previous
import jax
import jax.numpy as jnp
from jax import lax
from jax.experimental import pallas as pl
from jax.experimental.pallas import tpu as pltpu
from jax.experimental.pallas import tpu_sc as plsc

BATCH = 4096
SEQ = 500
EMB = 64
VOCAB = 1000

SEQ_PAD = 512
EROW = 128
NW = 32
B_PER_W = BATCH // NW
GROUP = 32
N_CHUNK = SEQ_PAD // 128


def _sc_body(data_hbm, e_hbm, x_hbm, espm, dbuf, xb0, xb1,
             semE, sg0, sg1, sw0, sw1):
    xbufs = (xb0, xb1)
    gsems = (sg0, sg1)
    wsems = (sw0, sw1)
    wid = lax.axis_index("s") * 2 + lax.axis_index("c")
    base = wid * B_PER_W

    @pl.when(lax.axis_index("s") == 0)
    def _():
        pltpu.async_copy(e_hbm, espm, semE).wait()

    plsc.subcore_barrier()

    def xsrc(c):
        return xbufs[c % 2]

    def xdst(b, c):
        return x_hbm.at[base + b, pl.ds(c * 128, 128)]

    def do_row(b, r, first):
        for c in range(N_CHUNK):
            if not (first and c < 2):
                pltpu.make_async_copy(xsrc(c), xdst(b, c),
                                      wsems[c % 2]).wait()
            pltpu.async_copy(espm.at[dbuf.at[r, c]], xbufs[c % 2],
                             gsems[c % 2]).wait()
            pltpu.async_copy(xsrc(c), xdst(b, c), wsems[c % 2])

    for g in range(B_PER_W // GROUP):
        pltpu.sync_copy(data_hbm.at[pl.ds(base + g * GROUP, GROUP)], dbuf)
        if g == 0:
            do_row(0, 0, first=True)
            lax.fori_loop(
                1, GROUP, lambda r, _: (do_row(r, r, False), 0)[1], 0)
        else:
            lax.fori_loop(
                0, GROUP,
                lambda r, _, gg=g: (do_row(gg * GROUP + r, r, False), 0)[1],
                0)
    pltpu.make_async_copy(xsrc(0), xdst(0, 0), wsems[0]).wait()
    pltpu.make_async_copy(xsrc(1), xdst(0, 1), wsems[1]).wait()


def _sc_gather(data_p, E_pad):
    mesh = plsc.VectorSubcoreMesh(core_axis_name="c", subcore_axis_name="s")
    f = pl.kernel(
        _sc_body,
        mesh=mesh,
        out_type=jax.ShapeDtypeStruct((BATCH, SEQ_PAD, EROW), jnp.float32),
        scratch_types=[
            pltpu.VMEM_SHARED((VOCAB, EROW), jnp.float32),
            pltpu.VMEM((GROUP, N_CHUNK, 128), jnp.int32),
            pltpu.VMEM((128, EROW), jnp.float32),
            pltpu.VMEM((128, EROW), jnp.float32),
            pltpu.SemaphoreType.DMA,
            pltpu.SemaphoreType.DMA,
            pltpu.SemaphoreType.DMA,
            pltpu.SemaphoreType.DMA,
            pltpu.SemaphoreType.DMA,
        ],
    )
    return f(data_p, E_pad)


B_TILE = 512
K_TILE = 4096


def _mlp_body(x_ref, w1_ref, b1_ref, w2_ref, b2_ref, w3_ref, b3_ref,
              mu_ref, sc_ref, acc_ref):
    k = pl.program_id(1)
    nk = pl.num_programs(1)

    @pl.when(k == 0)
    def _():
        acc_ref[...] = jnp.zeros_like(acc_ref)

    acc_ref[...] += jnp.dot(x_ref[...], w1_ref[...],
                            preferred_element_type=jnp.float32)

    @pl.when(k == nk - 1)
    def _():
        h1 = jnp.maximum(acc_ref[...] + b1_ref[...], 0.0)
        h2 = jnp.maximum(
            jnp.dot(h1, w2_ref[...], preferred_element_type=jnp.float32)
            + b2_ref[...], 0.0)
        out = jnp.dot(h2, w3_ref[...],
                      preferred_element_type=jnp.float32) + b3_ref[...]
        mu_ref[...] = out[:, :2]
        sc_ref[...] = jnp.logaddexp(out[:, 2:] - 5.0, 0.0)


def _mlp(x, W1p, b1, W2, b2, W3, b3):
    nb = BATCH // B_TILE
    nk = (SEQ_PAD * EROW) // K_TILE
    mu, scale = pl.pallas_call(
        _mlp_body,
        grid=(nb, nk),
        in_specs=[
            pl.BlockSpec((B_TILE, K_TILE), lambda b, k: (b, k)),
            pl.BlockSpec((K_TILE, 128), lambda b, k: (k, 0)),
            pl.BlockSpec((128,), lambda b, k: (0,)),
            pl.BlockSpec((128, 64), lambda b, k: (0, 0)),
            pl.BlockSpec((64,), lambda b, k: (0,)),
            pl.BlockSpec((64, 4), lambda b, k: (0, 0)),
            pl.BlockSpec((4,), lambda b, k: (0,)),
        ],
        out_specs=[
            pl.BlockSpec((B_TILE, 2), lambda b, k: (b, 0)),
            pl.BlockSpec((B_TILE, 2), lambda b, k: (b, 0)),
        ],
        out_shape=[
            jax.ShapeDtypeStruct((BATCH, 2), jnp.float32),
            jax.ShapeDtypeStruct((BATCH, 2), jnp.float32),
        ],
        scratch_shapes=[pltpu.VMEM((B_TILE, 128), jnp.float32)],
    )(x, W1p, b1, W2, b2, W3, b3)
    return mu, scale


def kernel(data, E, W1, b1, W2, b2, W3, b3):
    data_p = jnp.pad(data, ((0, 0), (0, SEQ_PAD - SEQ)))
    data_p = data_p.reshape(BATCH, N_CHUNK, 128)
    E_pad = jnp.pad(E, ((0, 0), (0, EROW - EMB)))
    x = _sc_gather(data_p, E_pad).reshape(BATCH, SEQ_PAD * EROW)
    W1pp = jnp.pad(W1.reshape(SEQ, EMB, 128),
                   ((0, SEQ_PAD - SEQ), (0, EROW - EMB), (0, 0)))
    W1pp = W1pp.reshape(SEQ_PAD * EROW, 128)
    return _mlp(x, W1pp, b1, W2, b2, W3, b3)


# speedup vs baseline: 21.1098x; 1.7846x over previous
import jax
import jax.numpy as jnp
from jax import lax
from jax.experimental import pallas as pl
from jax.experimental.pallas import tpu as pltpu
from jax.experimental.pallas import tpu_sc as plsc

BATCH = 4096
SEQ = 500
EMB = 64
VOCAB = 1000

SEQ_PAD = 512
EROW = 128
NW = 32
B_PER_W = BATCH // NW
GROUP = 32
N_CHUNK = SEQ_PAD // 128


def _sc_body(data_hbm, e_hbm, x_hbm, espm, dbuf, xb0, xb1,
             semE, sg0, sg1, sw0, sw1):
    xbufs = (xb0, xb1)
    gsems = (sg0, sg1)
    wsems = (sw0, sw1)
    wid = lax.axis_index("s") * 2 + lax.axis_index("c")
    base = wid * B_PER_W

    @pl.when(lax.axis_index("s") == 0)
    def _():
        pltpu.async_copy(e_hbm, espm, semE).wait()

    plsc.subcore_barrier()

    def xsrc(c):
        return xbufs[c % 2]

    def xdst(b, c):
        return x_hbm.at[base + b, pl.ds(c * 128, 128)]

    def do_row(b, r, first):
        for c in range(N_CHUNK):
            if not (first and c < 2):
                pltpu.make_async_copy(xsrc(c), xdst(b, c),
                                      wsems[c % 2]).wait()
            pltpu.async_copy(espm.at[dbuf.at[r, c]], xbufs[c % 2],
                             gsems[c % 2]).wait()
            pltpu.async_copy(xsrc(c), xdst(b, c), wsems[c % 2])

    for g in range(B_PER_W // GROUP):
        pltpu.sync_copy(data_hbm.at[pl.ds(base + g * GROUP, GROUP)], dbuf)
        if g == 0:
            do_row(0, 0, first=True)
            lax.fori_loop(
                1, GROUP, lambda r, _: (do_row(r, r, False), 0)[1], 0)
        else:
            lax.fori_loop(
                0, GROUP,
                lambda r, _, gg=g: (do_row(gg * GROUP + r, r, False), 0)[1],
                0)
    pltpu.make_async_copy(xsrc(0), xdst(0, 0), wsems[0]).wait()
    pltpu.make_async_copy(xsrc(1), xdst(0, 1), wsems[1]).wait()


def _sc_gather(data_p, E_pad):
    mesh = plsc.VectorSubcoreMesh(core_axis_name="c", subcore_axis_name="s")
    f = pl.kernel(
        _sc_body,
        mesh=mesh,
        out_type=jax.ShapeDtypeStruct((BATCH, SEQ_PAD, EROW), jnp.float32),
        scratch_types=[
            pltpu.VMEM_SHARED((VOCAB, EROW), jnp.float32),
            pltpu.VMEM((GROUP, N_CHUNK, 128), jnp.int32),
            pltpu.VMEM((128, EROW), jnp.float32),
            pltpu.VMEM((128, EROW), jnp.float32),
            pltpu.SemaphoreType.DMA,
            pltpu.SemaphoreType.DMA,
            pltpu.SemaphoreType.DMA,
            pltpu.SemaphoreType.DMA,
            pltpu.SemaphoreType.DMA,
        ],
    )
    return f(data_p, E_pad)


B_TILE = 512
S_CHUNK = 32


def _mlp_body(x_ref, w1_ref, b1_ref, w2_ref, b2_ref, w3_ref, b3_ref,
              mu_ref, sc_ref, acc_ref):
    k = pl.program_id(1)
    nk = pl.num_programs(1)

    @pl.when(k == 0)
    def _():
        acc_ref[...] = jnp.zeros_like(acc_ref)

    xb = x_ref[...].reshape(B_TILE, S_CHUNK * EROW)
    wb = w1_ref[...].reshape(S_CHUNK * EROW, 128)
    acc_ref[...] += jnp.dot(xb, wb, preferred_element_type=jnp.float32)

    @pl.when(k == nk - 1)
    def _():
        h1 = jnp.maximum(acc_ref[...] + b1_ref[...], 0.0)
        h2 = jnp.maximum(
            jnp.dot(h1, w2_ref[...], preferred_element_type=jnp.float32)
            + b2_ref[...], 0.0)
        out = jnp.dot(h2, w3_ref[...],
                      preferred_element_type=jnp.float32) + b3_ref[...]
        mu_ref[...] = out[:, :2]
        sc_ref[...] = jnp.logaddexp(out[:, 2:] - 5.0, 0.0)


def _mlp(x, W1p, b1, W2, b2, W3, b3):
    nb = BATCH // B_TILE
    nk = SEQ_PAD // S_CHUNK
    mu, scale = pl.pallas_call(
        _mlp_body,
        grid=(nb, nk),
        in_specs=[
            pl.BlockSpec((B_TILE, S_CHUNK, EROW), lambda b, k: (b, k, 0)),
            pl.BlockSpec((S_CHUNK, EROW, 128), lambda b, k: (k, 0, 0)),
            pl.BlockSpec((128,), lambda b, k: (0,)),
            pl.BlockSpec((128, 64), lambda b, k: (0, 0)),
            pl.BlockSpec((64,), lambda b, k: (0,)),
            pl.BlockSpec((64, 4), lambda b, k: (0, 0)),
            pl.BlockSpec((4,), lambda b, k: (0,)),
        ],
        out_specs=[
            pl.BlockSpec((B_TILE, 2), lambda b, k: (b, 0)),
            pl.BlockSpec((B_TILE, 2), lambda b, k: (b, 0)),
        ],
        out_shape=[
            jax.ShapeDtypeStruct((BATCH, 2), jnp.float32),
            jax.ShapeDtypeStruct((BATCH, 2), jnp.float32),
        ],
        scratch_shapes=[pltpu.VMEM((B_TILE, 128), jnp.float32)],
    )(x, W1p, b1, W2, b2, W3, b3)
    return mu, scale


def kernel(data, E, W1, b1, W2, b2, W3, b3):
    data_p = jnp.pad(data, ((0, 0), (0, SEQ_PAD - SEQ)))
    data_p = data_p.reshape(BATCH, N_CHUNK, 128)
    E_pad = jnp.pad(E, ((0, 0), (0, EROW - EMB)))
    x = _sc_gather(data_p, E_pad)
    W1pp = jnp.pad(W1.reshape(SEQ, EMB, 128),
                   ((0, SEQ_PAD - SEQ), (0, EROW - EMB), (0, 0)))
    return _mlp(x, W1pp, b1, W2, b2, W3, b3)
